# baseline (speedup 1.0000x reference)
import functools

import jax
import jax.numpy as jnp
import numpy as np
from jax import lax
from jax.experimental import pallas as pl
from jax.experimental.pallas import tpu as pltpu
from jax.experimental.pallas import tpu_sc as plsc

_N = 50000
_E = 800000
_K = 27
_C = 32
_B = 8

_R = 1000
_G = _N // _R
_NP = 50048
_ZR = 184
_NCH = 102
_CHB = 2
_EPAD = 32 * _NCH * _CHB * 128
_BLK = _EPAD // 128


def _sc_conv(xw_flat, f2d, dst2d):
    mesh = plsc.VectorSubcoreMesh(core_axis_name="c", subcore_axis_name="s")

    @functools.partial(
        pl.kernel, mesh=mesh,
        compiler_params=pltpu.CompilerParams(use_tc_tiling_on_sc=False),
        out_type=jax.ShapeDtypeStruct((2, _NP, _C), jnp.float32),
        scratch_types=[
            pltpu.VMEM((6, _CHB * 128), jnp.int32),
            pltpu.VMEM((6, _CHB * 128), jnp.int32),
            pltpu.VMEM((3, _CHB * 128, _C), jnp.float32),
            pltpu.VMEM_SHARED((_NP, _C), jnp.float32),
            pltpu.SemaphoreType.DMA,
            pltpu.SemaphoreType.DMA,
        ],
    )
    def body(xw_hbm, f_hbm, d_hbm, out_hbm, fi, di, rows, acc, gsem, ssem):
        cid = lax.axis_index("c")
        sid = lax.axis_index("s")

        def _zb(i, c):
            rows[0, i, pl.ds(0, 16)] = jnp.zeros((16,), jnp.float32)
            rows[0, i, pl.ds(16, 16)] = jnp.zeros((16,), jnp.float32)
            return c
        lax.fori_loop(0, _ZR, _zb, 0)
        zsrc = rows.at[0, pl.ds(0, _ZR)]

        def _zc(i, c):
            pltpu.async_copy(zsrc, acc.at[pl.ds(sid * 3128 + i * _ZR, _ZR)],
                             ssem)
            return c
        lax.fori_loop(0, 17, _zc, 0)

        def _zw(i, c):
            pltpu.make_async_copy(
                zsrc, acc.at[pl.ds(sid * 3128 + i * _ZR, _ZR)], ssem).wait()
            return c
        lax.fori_loop(0, 17, _zw, 0)
        plsc.subcore_barrier()

        base = (cid * 16 + sid) * (_NCH * _CHB)

        def load_idx(c, p):
            off = (base + c * _CHB) * 128
            pltpu.sync_copy(f_hbm.at[pl.ds(off, _CHB * 128)], fi.at[p])
            pltpu.sync_copy(d_hbm.at[pl.ds(off, _CHB * 128)], di.at[p])

        def gathers(fn, p2, p4):
            fn(xw_hbm.at[fi.at[p4]], rows.at[p2], gsem)

        def scatters(fn, p2, p4, **kw):
            fn(rows.at[p2], acc.at[di.at[p4]], ssem, **kw)

        def _issue(src, dst, sem, **kw):
            pltpu.async_copy(src, dst, sem, **kw)

        def _drain(src, dst, sem, **kw):
            pltpu.make_async_copy(src, dst, sem).wait()

        load_idx(0, 0)
        load_idx(1, 1)
        load_idx(2, 2)
        gathers(_issue, 0, 0)
        gathers(_issue, 1, 1)

        def _iter(i, carry):
            for k in range(6):
                c = i * 6 + k
                gathers(_drain, k % 3, k)
                if k == 0:
                    @pl.when(i >= 1)
                    def _():
                        scatters(_drain, 2, 5)
                else:
                    scatters(_drain, (k - 1) % 3, k - 1)

                @pl.when(c + 2 < _NCH)
                def _():
                    gathers(_issue, (k + 2) % 3, (k + 2) % 6)
                scatters(_issue, k % 3, k, add=True)

                @pl.when(c + 3 < _NCH)
                def _():
                    load_idx(c + 3, (k + 3) % 6)
            return carry
        lax.fori_loop(0, _NCH // 6, _iter, 0)
        scatters(_drain, 2, 5)
        plsc.subcore_barrier()

        def _co(i, c):
            r0 = sid * 3128 + i * _ZR
            pltpu.async_copy(acc.at[pl.ds(r0, _ZR)],
                             out_hbm.at[cid, pl.ds(r0, _ZR)], gsem)
            return c
        lax.fori_loop(0, 17, _co, 0)

        def _cw(i, c):
            r0 = sid * 3128 + i * _ZR
            pltpu.make_async_copy(acc.at[pl.ds(r0, _ZR)],
                                  out_hbm.at[cid, pl.ds(r0, _ZR)], gsem).wait()
            return c
        lax.fori_loop(0, 17, _cw, 0)

    return body(xw_flat, f2d, dst2d)


def _row_spec(shape_tail):
    return pl.BlockSpec((_R,) + shape_tail, lambda i: (i,) + (0,) * len(shape_tail))


def _row2(shape_tail):
    return pl.BlockSpec((_R,) + shape_tail,
                        lambda i, t: (i,) + (0,) * len(shape_tail))


def _p2():
    return pl.BlockSpec((2, _R, _C), lambda i, t: (0, i, 0))


def _full2(shape):
    return pl.BlockSpec(shape, lambda i, t: (0,) * len(shape))


def _w2spec(cin):
    return pl.BlockSpec((cin, 128), lambda i, t: (0, t))


def _xw2spec():
    return pl.BlockSpec((_R, 128), lambda i, t: (t * _G + i, 0))


def _mm2(body, ins, specs, w, want_h=False):
    nt = w.shape[1] // 128
    outs = [jax.ShapeDtypeStruct((nt * _N, 128), jnp.float32)]
    ospecs = [_xw2spec()]
    if want_h:
        outs.append(jax.ShapeDtypeStruct((_N, _C), jnp.float32))
        ospecs.append(_row2((_C,)))
    r = pl.pallas_call(
        body, grid=(_G, nt), in_specs=specs + [_w2spec(w.shape[0])],
        out_shape=outs, out_specs=ospecs)(*(ins + [w]))
    return r if want_h else r[0]


def _p_spec():
    return pl.BlockSpec((2, _R, _C), lambda i: (0, i, 0))


def _full_spec(shape):
    return pl.BlockSpec(shape, lambda i: (0,) * len(shape))


def _tc(body, ins, specs, out_shape, out_specs):
    return pl.pallas_call(
        body, grid=(_G,), in_specs=specs,
        out_shape=out_shape, out_specs=out_specs)(*ins)


def _mm_plain(x8, w):
    def body(x_ref, w_ref, xw_ref):
        xw_ref[...] = jnp.dot(x_ref[...], w_ref[...],
                              preferred_element_type=jnp.float32)
    return _mm2(body, [x8], [_row2((x8.shape[1],))], w)


def _bn_relu(pp, g, b):
    return jnp.maximum((pp[0] + pp[1]) * g + b, 0.0)


def _e1_mm(p, g, b, w, want_h=False):
    kc = w.shape[1]

    def body(p_ref, g_ref, b_ref, w_ref, xw_ref, *h_ref):
        h = _bn_relu(p_ref[...], g_ref[...], b_ref[...])
        xw_ref[...] = jnp.dot(h, w_ref[...], preferred_element_type=jnp.float32)
        if h_ref:
            h_ref[0][...] = h

    return _mm2(body, [p, g.reshape(1, _C), b.reshape(1, _C)],
                [_p2(), _full2((1, _C)), _full2((1, _C))], w, want_h)


def _e4_mm(p, g, b, w):
    kc = w.shape[1]

    def body(p_ref, g_ref, b_ref, w_ref, xw_ref):
        pp = p_ref[...]
        h = (pp[0] + pp[1]) * g_ref[...] + b_ref[...]
        xw_ref[...] = jnp.dot(h, w_ref[...], preferred_element_type=jnp.float32)

    return _mm2(body, [p, g.reshape(1, _C), b.reshape(1, _C)],
                [_p2(), _full2((1, _C)), _full2((1, _C))], w)


def _e2_mm(pa, pb, ga, ba, gb, bb, aw, w):
    kc = w.shape[1]

    def body(pa_ref, pb_ref, ga_r, ba_r, gb_r, bb_r, aw_r, w_ref, xw_ref, h_ref):
        s0 = jax.nn.sigmoid(aw_r[0])
        s1 = jax.nn.sigmoid(aw_r[1])
        oa = (pa_ref[0] + pa_ref[1]) * ga_r[...] + ba_r[...]
        ob = (pb_ref[0] + pb_ref[1]) * gb_r[...] + bb_r[...]
        h = jnp.maximum(s0 * oa + s1 * ob, 0.0)
        xw_ref[...] = jnp.dot(h, w_ref[...], preferred_element_type=jnp.float32)
        h_ref[...] = h

    return _mm2(body,
                [pa, pb, ga.reshape(1, _C), ba.reshape(1, _C),
                 gb.reshape(1, _C), bb.reshape(1, _C), aw.reshape(2, 1, 1)],
                [_p2(), _p2(), _full2((1, _C)), _full2((1, _C)),
                 _full2((1, _C)), _full2((1, _C)), _full2((2, 1, 1))],
                w, want_h=True)


def _e3_mm(p, g, b, skip, w):
    kc = w.shape[1]

    def body(p_ref, g_ref, b_ref, s_ref, w_ref, xw_ref, h_ref):
        pp = p_ref[...]
        h = jnp.maximum((pp[0] + pp[1]) * g_ref[...] + b_ref[...] + s_ref[...],
                        0.0)
        xw_ref[...] = jnp.dot(h, w_ref[...], preferred_element_type=jnp.float32)
        h_ref[...] = h

    return _mm2(body, [p, g.reshape(1, _C), b.reshape(1, _C), skip],
                [_p2(), _full2((1, _C)), _full2((1, _C)), _row2((_C,))],
                w, want_h=True)


def _e1_only(p, g, b):
    def body(p_ref, g_ref, b_ref, h_ref):
        h_ref[...] = _bn_relu(p_ref[...], g_ref[...], b_ref[...])

    return _tc(body, [p, g.reshape(1, _C), b.reshape(1, _C)],
               [_p_spec(), _full_spec((1, _C)), _full_spec((1, _C))],
               jax.ShapeDtypeStruct((_N, _C), jnp.float32), _row_spec((_C,)))


def _seg_reduce(c0, c1, c2, b2d):
    def body(c0_r, c1_r, c2_r, b_r, s0, s1, s2, m0, m1, m2, cnt):
        i = pl.program_id(0)
        oh = (b_r[...] == lax.broadcasted_iota(jnp.int32, (1, _B), 1)
              ).astype(jnp.float32)
        dn = (((0,), (0,)), ((), ()))
        cs = [c0_r[...], c1_r[...], c2_r[...]]
        ss = [lax.dot_general(oh, c, dn, preferred_element_type=jnp.float32)
              for c in cs]
        cn = lax.dot_general(oh, jnp.ones((_R, 1), jnp.float32), dn,
                             preferred_element_type=jnp.float32)
        ms = []
        for c in cs:
            rows = []
            for bb in range(_B):
                msk = b_r[...] == bb
                rows.append(jnp.max(jnp.where(msk, c, -jnp.inf), axis=0,
                                    keepdims=True))
            ms.append(jnp.concatenate(rows, axis=0))

        @pl.when(i == 0)
        def _init():
            for r, v in zip((s0, s1, s2), ss):
                r[...] = v
            for r, v in zip((m0, m1, m2), ms):
                r[...] = v
            cnt[...] = cn

        @pl.when(i > 0)
        def _acc():
            for r, v in zip((s0, s1, s2), ss):
                r[...] += v
            for r, v in zip((m0, m1, m2), ms):
                r[...] = jnp.maximum(r[...], v)
            cnt[...] += cn

    o83 = jax.ShapeDtypeStruct((_B, _C), jnp.float32)
    return _tc(body, [c0, c1, c2, b2d],
               [_row_spec((_C,))] * 3 + [_row_spec((1,))],
               [o83] * 6 + [jax.ShapeDtypeStruct((_B, 1), jnp.float32)],
               [_full_spec((_B, _C))] * 6 + [_full_spec((_B, 1))])


def _final(c0, c1, c2, h4, b2d, reds, fc1r, fb1, fc2r, fb2r, ksw, pmats):
    s0, s1, s2, m0, m1, m2, cnt = reds

    def body(c0_r, c1_r, c2_r, h4_r, b_r, s0_r, s1_r, s2_r, m0_r, m1_r, m2_r,
             cnt_r, fc1_r, fb1_r, fc2_r, fb2_r, ksw_r, pm_r, out_ref):
        cn = jnp.maximum(cnt_r[...], 1.0)
        avgs = [s0_r[...] / cn, s1_r[...] / cn, s2_r[...] / cn]
        mxs = [jnp.where(m_r[...] == -jnp.inf, 0.0, m_r[...])
               for m_r in (m0_r, m1_r, m2_r)]
        za = jnp.zeros((_B, 6), jnp.float32)
        zm = jnp.zeros((_B, 6), jnp.float32)
        for m in range(3):
            za = za + jnp.dot(avgs[m], fc1_r[m],
                              preferred_element_type=jnp.float32)
            zm = zm + jnp.dot(mxs[m], fc1_r[m],
                              preferred_element_type=jnp.float32)
        ha = jnp.maximum(za + fb1_r[...], 0.0)
        hm = jnp.maximum(zm + fb1_r[...], 0.0)
        hh = ha + hm
        atts = [jax.nn.sigmoid(jnp.dot(hh, fc2_r[m],
                                       preferred_element_type=jnp.float32)
                               + 2.0 * fb2_r[m])
                for m in range(3)]
        oh = (b_r[...] == lax.broadcasted_iota(jnp.int32, (1, _B), 1)
              ).astype(jnp.float32)
        cs = [c0_r[...], c1_r[...], c2_r[...]]
        oas = [cs[m] * jnp.dot(oh, atts[m], preferred_element_type=jnp.float32)
               for m in range(3)]
        out = h4_r[...]
        for j in range(3):
            mj = jnp.zeros((_R, _C), jnp.float32)
            for m in range(3):
                mj = mj + jnp.dot(oas[m], pm_r[j, m],
                                  preferred_element_type=jnp.float32)
            w3j = jax.nn.sigmoid(ksw_r[j])
            out = out + w3j * (cs[j] + mj)
        out_ref[...] = jnp.maximum(out, 0.0)

    return _tc(body,
               [c0, c1, c2, h4, b2d, s0, s1, s2, m0, m1, m2, cnt,
                fc1r, fb1.reshape(1, 6), fc2r, fb2r, ksw.reshape(3, 1, 1),
                pmats],
               [_row_spec((_C,))] * 4 + [_row_spec((1,))]
               + [_full_spec((_B, _C))] * 6 + [_full_spec((_B, 1))]
               + [_full_spec((3, _C, 6)), _full_spec((1, 6)),
                  _full_spec((3, 6, _C)), _full_spec((3, 1, _C)),
                  _full_spec((3, 1, 1)), _full_spec((3, 3, _C, _C))],
               jax.ShapeDtypeStruct((_N, _C), jnp.float32), _row_spec((_C,)))


def _w2d(W):
    cin = W.shape[1]
    w2 = W.transpose(1, 0, 2).reshape(cin, _K * _C)
    return jnp.concatenate([w2, jnp.zeros((cin, _C), jnp.float32)], axis=1)


_PM = np.zeros((3, 3, _C, _C), np.float32)
for _j in range(3):
    for _cc in range(_C):
        _t = 3 * _cc + _j
        _PM[_j, _t // _C, _t % _C, _cc] = 1.0


def kernel(x, params, edge_index, kernel_offsets, batch_idx):
    p = params
    src = edge_index[0].astype(jnp.int32)
    dst = edge_index[1].astype(jnp.int32)
    ko = kernel_offsets.astype(jnp.int32)
    pad = _EPAD - _E

    def _pad2d(a, fill):
        return jnp.concatenate([a, jnp.full((pad,), fill, jnp.int32)])

    fbase = (ko >> 2) * (4 * _N) + src * 4 + (ko & 3)
    f27 = _pad2d(fbase, 0)
    dstp = _pad2d(dst, _N)
    f54a = f27
    f54b = _pad2d(fbase + 28 * _N, 0)
    b2d = batch_idx.astype(jnp.int32).reshape(_N, 1)

    x8 = jnp.concatenate([x, jnp.zeros((_N, 5), jnp.float32)], axis=1)
    ws1 = jnp.concatenate([_w2d(p['Ws1']),
                           jnp.zeros((5, 28 * _C), jnp.float32)], axis=0)

    def conv(xw, f):
        return _sc_conv(xw.reshape(-1, _C), f, dstp)

    xw = _mm_plain(x8, ws1)
    pp = conv(xw, f27)
    xw = _e1_mm(pp, p['gs1'], p['bs1'], _w2d(p['Ws2']))
    pp = conv(xw, f27)
    wa = jnp.concatenate([_w2d(p['Wa2']), _w2d(p['Wa3'])], axis=1)
    xw2 = _e1_mm(pp, p['gs2'], p['bs2'], wa)
    pa = conv(xw2, f54a)
    pb = conv(xw2, f54b)
    xw, h3 = _e2_mm(pa, pb, p['ga2'], p['ba2'], p['ga3'], p['ba3'],
                    p['adff_w'], _w2d(p['Wr1']))
    pp = conv(xw, f27)
    xw = _e1_mm(pp, p['gr1'], p['br1'], _w2d(p['Wr2']))
    pp = conv(xw, f27)
    xw, h4 = _e3_mm(pp, p['gr2'], p['br2'], h3, _w2d(p['Wg1']))
    pp = conv(xw, f27)
    xw = _e1_mm(pp, p['gg1'], p['bg1'], _w2d(p['Wg2']))
    pp = conv(xw, f27)
    xw = _e4_mm(pp, p['gg2'], p['bg2'], _w2d(p['Wk0']))
    pp = conv(xw, f27)
    xw, c0 = _e1_mm(pp, p['gk0'], p['bk0'], _w2d(p['Wk1']), want_h=True)
    pp = conv(xw, f27)
    xw, c1 = _e1_mm(pp, p['gk1'], p['bk1'], _w2d(p['Wk2']), want_h=True)
    pp = conv(xw, f27)
    c2 = _e1_only(pp, p['gk2'], p['bk2'])

    reds = _seg_reduce(c0, c1, c2, b2d)
    fc1r = p['fc1'].reshape(3, _C, 6)
    fc2r = p['fc2'].reshape(6, 3, _C).transpose(1, 0, 2)
    fb2r = p['fb2'].reshape(1, 3, _C).transpose(1, 0, 2)
    return _final(c0, c1, c2, h4, b2d, reds, fc1r, p['fb1'], fc2r, fb2r,
                  p['ksw'], jnp.asarray(_PM))


# speedup vs baseline: 1.0038x; 1.0038x over previous
import functools

import jax
import jax.numpy as jnp
import numpy as np
from jax import lax
from jax.experimental import pallas as pl
from jax.experimental.pallas import tpu as pltpu
from jax.experimental.pallas import tpu_sc as plsc

_N = 50000
_E = 800000
_K = 27
_C = 32
_B = 8

_R = 1000
_G = _N // _R
_NP = 50048
_ZR = 184
_NCH = 102
_CHB = 2
_EPAD = 32 * _NCH * _CHB * 128
_BLK = _EPAD // 128


def _sc_conv(xw_flat, f2d, dst2d):
    mesh = plsc.VectorSubcoreMesh(core_axis_name="c", subcore_axis_name="s")

    @functools.partial(
        pl.kernel, mesh=mesh,
        compiler_params=pltpu.CompilerParams(use_tc_tiling_on_sc=False),
        out_type=jax.ShapeDtypeStruct((2, _NP, _C), jnp.float32),
        scratch_types=[
            pltpu.VMEM((6, _CHB * 128), jnp.int32),
            pltpu.VMEM((6, _CHB * 128), jnp.int32),
            pltpu.VMEM((3, _CHB * 128, _C), jnp.float32),
            pltpu.VMEM_SHARED((_NP, _C), jnp.float32),
            pltpu.SemaphoreType.DMA,
            pltpu.SemaphoreType.DMA,
        ],
    )
    def body(xw_hbm, f_hbm, d_hbm, out_hbm, fi, di, rows, acc, gsem, ssem):
        cid = lax.axis_index("c")
        sid = lax.axis_index("s")

        def _zb(i, c):
            rows[2, i, pl.ds(0, 16)] = jnp.zeros((16,), jnp.float32)
            rows[2, i, pl.ds(16, 16)] = jnp.zeros((16,), jnp.float32)
            return c
        lax.fori_loop(0, _ZR, _zb, 0)
        zsrc = rows.at[2, pl.ds(0, _ZR)]

        def _zc(i, c):
            pltpu.async_copy(zsrc, acc.at[pl.ds(sid * 3128 + i * _ZR, _ZR)],
                             ssem)
            return c
        lax.fori_loop(0, 17, _zc, 0)

        base = (cid * 16 + sid) * (_NCH * _CHB)

        def load_idx(c, p):
            off = (base + c * _CHB) * 128
            pltpu.sync_copy(f_hbm.at[pl.ds(off, _CHB * 128)], fi.at[p])
            pltpu.sync_copy(d_hbm.at[pl.ds(off, _CHB * 128)], di.at[p])

        def gathers(fn, p2, p4):
            fn(xw_hbm.at[fi.at[p4]], rows.at[p2], gsem)

        def scatters(fn, p2, p4, **kw):
            fn(rows.at[p2], acc.at[di.at[p4]], ssem, **kw)

        def _issue(src, dst, sem, **kw):
            pltpu.async_copy(src, dst, sem, **kw)

        def _drain(src, dst, sem, **kw):
            pltpu.make_async_copy(src, dst, sem).wait()

        load_idx(0, 0)
        load_idx(1, 1)
        load_idx(2, 2)
        gathers(_issue, 0, 0)
        gathers(_issue, 1, 1)

        def _zw(i, c):
            pltpu.make_async_copy(
                zsrc, acc.at[pl.ds(sid * 3128 + i * _ZR, _ZR)], ssem).wait()
            return c
        lax.fori_loop(0, 17, _zw, 0)
        plsc.subcore_barrier()

        def _iter(i, carry):
            for k in range(6):
                c = i * 6 + k
                gathers(_drain, k % 3, k)
                if k == 0:
                    @pl.when(i >= 1)
                    def _():
                        scatters(_drain, 2, 5)
                else:
                    scatters(_drain, (k - 1) % 3, k - 1)

                @pl.when(c + 2 < _NCH)
                def _():
                    gathers(_issue, (k + 2) % 3, (k + 2) % 6)
                scatters(_issue, k % 3, k, add=True)

                @pl.when(c + 3 < _NCH)
                def _():
                    load_idx(c + 3, (k + 3) % 6)
            return carry
        lax.fori_loop(0, _NCH // 6, _iter, 0)
        scatters(_drain, 2, 5)
        plsc.subcore_barrier()

        def _co(i, c):
            r0 = sid * 3128 + i * _ZR
            pltpu.async_copy(acc.at[pl.ds(r0, _ZR)],
                             out_hbm.at[cid, pl.ds(r0, _ZR)], gsem)
            return c
        lax.fori_loop(0, 17, _co, 0)

        def _cw(i, c):
            r0 = sid * 3128 + i * _ZR
            pltpu.make_async_copy(acc.at[pl.ds(r0, _ZR)],
                                  out_hbm.at[cid, pl.ds(r0, _ZR)], gsem).wait()
            return c
        lax.fori_loop(0, 17, _cw, 0)

    return body(xw_flat, f2d, dst2d)


def _row_spec(shape_tail):
    return pl.BlockSpec((_R,) + shape_tail, lambda i: (i,) + (0,) * len(shape_tail))


def _row2(shape_tail):
    return pl.BlockSpec((_R,) + shape_tail,
                        lambda i, t: (i,) + (0,) * len(shape_tail))


def _p2():
    return pl.BlockSpec((2, _R, _C), lambda i, t: (0, i, 0))


def _full2(shape):
    return pl.BlockSpec(shape, lambda i, t: (0,) * len(shape))


def _w2spec(cin):
    return pl.BlockSpec((cin, 128), lambda i, t: (0, t))


def _xw2spec():
    return pl.BlockSpec((_R, 128), lambda i, t: (t * _G + i, 0))


def _mm2(body, ins, specs, w, want_h=False):
    nt = w.shape[1] // 128
    outs = [jax.ShapeDtypeStruct((nt * _N, 128), jnp.float32)]
    ospecs = [_xw2spec()]
    if want_h:
        outs.append(jax.ShapeDtypeStruct((_N, _C), jnp.float32))
        ospecs.append(_row2((_C,)))
    r = pl.pallas_call(
        body, grid=(_G, nt), in_specs=specs + [_w2spec(w.shape[0])],
        out_shape=outs, out_specs=ospecs)(*(ins + [w]))
    return r if want_h else r[0]


def _p_spec():
    return pl.BlockSpec((2, _R, _C), lambda i: (0, i, 0))


def _full_spec(shape):
    return pl.BlockSpec(shape, lambda i: (0,) * len(shape))


def _tc(body, ins, specs, out_shape, out_specs):
    return pl.pallas_call(
        body, grid=(_G,), in_specs=specs,
        out_shape=out_shape, out_specs=out_specs)(*ins)


def _mm_plain(x8, w):
    def body(x_ref, w_ref, xw_ref):
        xw_ref[...] = jnp.dot(x_ref[...], w_ref[...],
                              preferred_element_type=jnp.float32)
    return _mm2(body, [x8], [_row2((x8.shape[1],))], w)


def _bn_relu(pp, g, b):
    return jnp.maximum((pp[0] + pp[1]) * g + b, 0.0)


def _e1_mm(p, g, b, w, want_h=False):
    kc = w.shape[1]

    def body(p_ref, g_ref, b_ref, w_ref, xw_ref, *h_ref):
        h = _bn_relu(p_ref[...], g_ref[...], b_ref[...])
        xw_ref[...] = jnp.dot(h, w_ref[...], preferred_element_type=jnp.float32)
        if h_ref:
            h_ref[0][...] = h

    return _mm2(body, [p, g.reshape(1, _C), b.reshape(1, _C)],
                [_p2(), _full2((1, _C)), _full2((1, _C))], w, want_h)


def _e4_mm(p, g, b, w):
    kc = w.shape[1]

    def body(p_ref, g_ref, b_ref, w_ref, xw_ref):
        pp = p_ref[...]
        h = (pp[0] + pp[1]) * g_ref[...] + b_ref[...]
        xw_ref[...] = jnp.dot(h, w_ref[...], preferred_element_type=jnp.float32)

    return _mm2(body, [p, g.reshape(1, _C), b.reshape(1, _C)],
                [_p2(), _full2((1, _C)), _full2((1, _C))], w)


def _e2_mm(pa, pb, ga, ba, gb, bb, aw, w):
    kc = w.shape[1]

    def body(pa_ref, pb_ref, ga_r, ba_r, gb_r, bb_r, aw_r, w_ref, xw_ref, h_ref):
        s0 = jax.nn.sigmoid(aw_r[0])
        s1 = jax.nn.sigmoid(aw_r[1])
        oa = (pa_ref[0] + pa_ref[1]) * ga_r[...] + ba_r[...]
        ob = (pb_ref[0] + pb_ref[1]) * gb_r[...] + bb_r[...]
        h = jnp.maximum(s0 * oa + s1 * ob, 0.0)
        xw_ref[...] = jnp.dot(h, w_ref[...], preferred_element_type=jnp.float32)
        h_ref[...] = h

    return _mm2(body,
                [pa, pb, ga.reshape(1, _C), ba.reshape(1, _C),
                 gb.reshape(1, _C), bb.reshape(1, _C), aw.reshape(2, 1, 1)],
                [_p2(), _p2(), _full2((1, _C)), _full2((1, _C)),
                 _full2((1, _C)), _full2((1, _C)), _full2((2, 1, 1))],
                w, want_h=True)


def _e3_mm(p, g, b, skip, w):
    kc = w.shape[1]

    def body(p_ref, g_ref, b_ref, s_ref, w_ref, xw_ref, h_ref):
        pp = p_ref[...]
        h = jnp.maximum((pp[0] + pp[1]) * g_ref[...] + b_ref[...] + s_ref[...],
                        0.0)
        xw_ref[...] = jnp.dot(h, w_ref[...], preferred_element_type=jnp.float32)
        h_ref[...] = h

    return _mm2(body, [p, g.reshape(1, _C), b.reshape(1, _C), skip],
                [_p2(), _full2((1, _C)), _full2((1, _C)), _row2((_C,))],
                w, want_h=True)


def _e1_only(p, g, b):
    def body(p_ref, g_ref, b_ref, h_ref):
        h_ref[...] = _bn_relu(p_ref[...], g_ref[...], b_ref[...])

    return _tc(body, [p, g.reshape(1, _C), b.reshape(1, _C)],
               [_p_spec(), _full_spec((1, _C)), _full_spec((1, _C))],
               jax.ShapeDtypeStruct((_N, _C), jnp.float32), _row_spec((_C,)))


def _seg_reduce(c0, c1, c2, b2d):
    def body(c0_r, c1_r, c2_r, b_r, s0, s1, s2, m0, m1, m2, cnt):
        i = pl.program_id(0)
        oh = (b_r[...] == lax.broadcasted_iota(jnp.int32, (1, _B), 1)
              ).astype(jnp.float32)
        dn = (((0,), (0,)), ((), ()))
        cs = [c0_r[...], c1_r[...], c2_r[...]]
        ss = [lax.dot_general(oh, c, dn, preferred_element_type=jnp.float32)
              for c in cs]
        cn = lax.dot_general(oh, jnp.ones((_R, 1), jnp.float32), dn,
                             preferred_element_type=jnp.float32)
        ms = []
        for c in cs:
            rows = []
            for bb in range(_B):
                msk = b_r[...] == bb
                rows.append(jnp.max(jnp.where(msk, c, -jnp.inf), axis=0,
                                    keepdims=True))
            ms.append(jnp.concatenate(rows, axis=0))

        @pl.when(i == 0)
        def _init():
            for r, v in zip((s0, s1, s2), ss):
                r[...] = v
            for r, v in zip((m0, m1, m2), ms):
                r[...] = v
            cnt[...] = cn

        @pl.when(i > 0)
        def _acc():
            for r, v in zip((s0, s1, s2), ss):
                r[...] += v
            for r, v in zip((m0, m1, m2), ms):
                r[...] = jnp.maximum(r[...], v)
            cnt[...] += cn

    o83 = jax.ShapeDtypeStruct((_B, _C), jnp.float32)
    return _tc(body, [c0, c1, c2, b2d],
               [_row_spec((_C,))] * 3 + [_row_spec((1,))],
               [o83] * 6 + [jax.ShapeDtypeStruct((_B, 1), jnp.float32)],
               [_full_spec((_B, _C))] * 6 + [_full_spec((_B, 1))])


def _final(c0, c1, c2, h4, b2d, reds, fc1r, fb1, fc2r, fb2r, ksw, pmats):
    s0, s1, s2, m0, m1, m2, cnt = reds

    def body(c0_r, c1_r, c2_r, h4_r, b_r, s0_r, s1_r, s2_r, m0_r, m1_r, m2_r,
             cnt_r, fc1_r, fb1_r, fc2_r, fb2_r, ksw_r, pm_r, out_ref):
        cn = jnp.maximum(cnt_r[...], 1.0)
        avgs = [s0_r[...] / cn, s1_r[...] / cn, s2_r[...] / cn]
        mxs = [jnp.where(m_r[...] == -jnp.inf, 0.0, m_r[...])
               for m_r in (m0_r, m1_r, m2_r)]
        za = jnp.zeros((_B, 6), jnp.float32)
        zm = jnp.zeros((_B, 6), jnp.float32)
        for m in range(3):
            za = za + jnp.dot(avgs[m], fc1_r[m],
                              preferred_element_type=jnp.float32)
            zm = zm + jnp.dot(mxs[m], fc1_r[m],
                              preferred_element_type=jnp.float32)
        ha = jnp.maximum(za + fb1_r[...], 0.0)
        hm = jnp.maximum(zm + fb1_r[...], 0.0)
        hh = ha + hm
        atts = [jax.nn.sigmoid(jnp.dot(hh, fc2_r[m],
                                       preferred_element_type=jnp.float32)
                               + 2.0 * fb2_r[m])
                for m in range(3)]
        oh = (b_r[...] == lax.broadcasted_iota(jnp.int32, (1, _B), 1)
              ).astype(jnp.float32)
        cs = [c0_r[...], c1_r[...], c2_r[...]]
        oas = [cs[m] * jnp.dot(oh, atts[m], preferred_element_type=jnp.float32)
               for m in range(3)]
        out = h4_r[...]
        for j in range(3):
            mj = jnp.zeros((_R, _C), jnp.float32)
            for m in range(3):
                mj = mj + jnp.dot(oas[m], pm_r[j, m],
                                  preferred_element_type=jnp.float32)
            w3j = jax.nn.sigmoid(ksw_r[j])
            out = out + w3j * (cs[j] + mj)
        out_ref[...] = jnp.maximum(out, 0.0)

    return _tc(body,
               [c0, c1, c2, h4, b2d, s0, s1, s2, m0, m1, m2, cnt,
                fc1r, fb1.reshape(1, 6), fc2r, fb2r, ksw.reshape(3, 1, 1),
                pmats],
               [_row_spec((_C,))] * 4 + [_row_spec((1,))]
               + [_full_spec((_B, _C))] * 6 + [_full_spec((_B, 1))]
               + [_full_spec((3, _C, 6)), _full_spec((1, 6)),
                  _full_spec((3, 6, _C)), _full_spec((3, 1, _C)),
                  _full_spec((3, 1, 1)), _full_spec((3, 3, _C, _C))],
               jax.ShapeDtypeStruct((_N, _C), jnp.float32), _row_spec((_C,)))


def _w2d(W):
    cin = W.shape[1]
    w2 = W.transpose(1, 0, 2).reshape(cin, _K * _C)
    return jnp.concatenate([w2, jnp.zeros((cin, _C), jnp.float32)], axis=1)


_PM = np.zeros((3, 3, _C, _C), np.float32)
for _j in range(3):
    for _cc in range(_C):
        _t = 3 * _cc + _j
        _PM[_j, _t // _C, _t % _C, _cc] = 1.0


def kernel(x, params, edge_index, kernel_offsets, batch_idx):
    p = params
    src = edge_index[0].astype(jnp.int32)
    dst = edge_index[1].astype(jnp.int32)
    ko = kernel_offsets.astype(jnp.int32)
    pad = _EPAD - _E

    def _pad2d(a, fill):
        return jnp.concatenate([a, jnp.full((pad,), fill, jnp.int32)])

    fbase = (ko >> 2) * (4 * _N) + src * 4 + (ko & 3)
    f27 = _pad2d(fbase, 0)
    dstp = _pad2d(dst, _N)
    f54a = f27
    f54b = _pad2d(fbase + 28 * _N, 0)
    b2d = batch_idx.astype(jnp.int32).reshape(_N, 1)

    x8 = jnp.concatenate([x, jnp.zeros((_N, 5), jnp.float32)], axis=1)
    ws1 = jnp.concatenate([_w2d(p['Ws1']),
                           jnp.zeros((5, 28 * _C), jnp.float32)], axis=0)

    def conv(xw, f):
        return _sc_conv(xw.reshape(-1, _C), f, dstp)

    xw = _mm_plain(x8, ws1)
    pp = conv(xw, f27)
    xw = _e1_mm(pp, p['gs1'], p['bs1'], _w2d(p['Ws2']))
    pp = conv(xw, f27)
    wa = jnp.concatenate([_w2d(p['Wa2']), _w2d(p['Wa3'])], axis=1)
    xw2 = _e1_mm(pp, p['gs2'], p['bs2'], wa)
    pa = conv(xw2, f54a)
    pb = conv(xw2, f54b)
    xw, h3 = _e2_mm(pa, pb, p['ga2'], p['ba2'], p['ga3'], p['ba3'],
                    p['adff_w'], _w2d(p['Wr1']))
    pp = conv(xw, f27)
    xw = _e1_mm(pp, p['gr1'], p['br1'], _w2d(p['Wr2']))
    pp = conv(xw, f27)
    xw, h4 = _e3_mm(pp, p['gr2'], p['br2'], h3, _w2d(p['Wg1']))
    pp = conv(xw, f27)
    xw = _e1_mm(pp, p['gg1'], p['bg1'], _w2d(p['Wg2']))
    pp = conv(xw, f27)
    xw = _e4_mm(pp, p['gg2'], p['bg2'], _w2d(p['Wk0']))
    pp = conv(xw, f27)
    xw, c0 = _e1_mm(pp, p['gk0'], p['bk0'], _w2d(p['Wk1']), want_h=True)
    pp = conv(xw, f27)
    xw, c1 = _e1_mm(pp, p['gk1'], p['bk1'], _w2d(p['Wk2']), want_h=True)
    pp = conv(xw, f27)
    c2 = _e1_only(pp, p['gk2'], p['bk2'])

    reds = _seg_reduce(c0, c1, c2, b2d)
    fc1r = p['fc1'].reshape(3, _C, 6)
    fc2r = p['fc2'].reshape(6, 3, _C).transpose(1, 0, 2)
    fb2r = p['fb2'].reshape(1, 3, _C).transpose(1, 0, 2)
    return _final(c0, c1, c2, h4, b2d, reds, fc1r, p['fb1'], fc2r, fb2r,
                  p['ksw'], jnp.asarray(_PM))
